# baseline (speedup 1.0000x reference)
import jax
import jax.numpy as jnp
from jax.experimental import pallas as pl
from jax.experimental.pallas import tpu as pltpu

_BK = 512


def _body(x_ref, L_ref, w_ref, b_ref, out_ref, y_ref):
    n, d = x_ref.shape
    k = pl.program_id(0)

    @pl.when(k == 0)
    def _():
        y_ref[...] = jax.lax.dot_general(
            x_ref[...], w_ref[:, d:],
            (((1,), (1,)), ((), ())),
            preferred_element_type=jnp.float32)
        out_ref[...] = jax.lax.dot_general(
            x_ref[...], w_ref[:, :d],
            (((1,), (1,)), ((), ())),
            preferred_element_type=jnp.float32) + b_ref[...]

    out_ref[...] += jax.lax.dot_general(
        L_ref[...], y_ref[pl.ds(k * _BK, _BK), :],
        (((1,), (0,)), ((), ())),
        preferred_element_type=jnp.float32)


def kernel(L, x, W, b):
    n, d = x.shape
    out_dim = W.shape[0]
    b2 = b.reshape(1, out_dim)
    return pl.pallas_call(
        _body,
        grid=(n // _BK,),
        in_specs=[
            pl.BlockSpec((n, d), lambda k: (0, 0)),
            pl.BlockSpec((n, _BK), lambda k: (0, k)),
            pl.BlockSpec((out_dim, 2 * d), lambda k: (0, 0)),
            pl.BlockSpec((1, out_dim), lambda k: (0, 0)),
        ],
        out_specs=pl.BlockSpec((n, out_dim), lambda k: (0, 0)),
        out_shape=jax.ShapeDtypeStruct((n, out_dim), jnp.float32),
        scratch_shapes=[pltpu.VMEM((n, out_dim), jnp.float32)],
        compiler_params=pltpu.CompilerParams(
            dimension_semantics=("arbitrary",),
        ),
    )(x, L, W, b2)
